# Initial kernel scaffold; baseline (speedup 1.0000x reference)
import jax
import jax.numpy as jnp
from jax.experimental import pallas as pl


def kernel(x, edge_index, batch, W1, a1_src, a1_dst, b1, W2, a2_src, a2_dst, b2, W_mu, b_mu, W_sigma, b_sigma):
    raise NotImplementedError("write your pallas kernel here")


# speedup vs baseline: 22.3318x; 22.3318x over previous
import functools

import jax
import jax.numpy as jnp
from jax import lax
from jax.experimental import pallas as pl
from jax.experimental.pallas import tpu as pltpu
from jax.experimental.pallas import tpu_sc as plsc

N = 10000
NP = 10240
E = 320000
D_IN = 128
D_H = 64
OUT_DIM = 14
SEQ_OUT = 12
NGRAPH = 32

NT = 32
EPT = E // NT
NCHUNK = 125
CH = EPT // NCHUNK
SEG = NP // 16

_f32 = jnp.float32


def _lrelu(x):
    return jnp.maximum(x, 0.2 * x)


def _tc1_body(x_ref, w1_ref, a1s_ref, a1d_ref, h_ref, as_ref, ad_ref, mb_ref):
    h = jnp.dot(x_ref[...], w1_ref[...], preferred_element_type=_f32)
    h_ref[...] = h
    a_s = jnp.dot(h, a1s_ref[...], preferred_element_type=_f32)
    as_ref[...] = a_s
    ad_ref[...] = jnp.dot(h, a1d_ref[...], preferred_element_type=_f32)
    mb_ref[...] = jnp.full((128,), jnp.max(a_s), _f32)


def _agg(osum_ref, den_ref, h_ref, as_ref, ad_ref, b_ref):
    a_s = as_ref[...]
    a_d = ad_ref[...]
    m = jnp.max(a_s)
    ws = jnp.exp(_lrelu(a_s + a_d) - _lrelu(m + a_d))
    den = den_ref[0] + den_ref[1] + ws
    osum = osum_ref[0] + osum_ref[1] + ws[:, None] * h_ref[...]
    return jnp.maximum(osum / den[:, None] + b_ref[...], 0.0)


def _tc2_body(osum_ref, den_ref, h_ref, as_ref, ad_ref, b_ref, w2_ref,
              a2s_ref, a2d_ref, h2_ref, as2_ref, ad2_ref, mb_ref):
    hr = _agg(osum_ref, den_ref, h_ref, as_ref, ad_ref, b_ref)
    h2 = jnp.dot(hr, w2_ref[...], preferred_element_type=_f32)
    h2_ref[...] = h2
    a_s2 = jnp.dot(h2, a2s_ref[...], preferred_element_type=_f32)
    as2_ref[...] = a_s2
    ad2_ref[...] = jnp.dot(h2, a2d_ref[...], preferred_element_type=_f32)
    mb_ref[...] = jnp.full((128,), jnp.max(a_s2), _f32)


def _tc3_body(osum_ref, den_ref, h_ref, as_ref, ad_ref, b_ref, batch_ref,
              wmu_ref, bmu_ref, wsg_ref, bsg_ref, mu_ref, sg_ref):
    hr = _agg(osum_ref, den_ref, h_ref, as_ref, ad_ref, b_ref)
    gids = lax.broadcasted_iota(jnp.int32, (NP, NGRAPH), 1)
    oh = (batch_ref[...][:, None] == gids).astype(_f32)
    gsum = lax.dot_general(oh, hr, (((0,), (0,)), ((), ())),
                           preferred_element_type=_f32)
    cnt = jnp.sum(oh, axis=0)
    g = gsum / jnp.maximum(cnt, 1.0)[:, None]
    mu_ref[...] = jnp.dot(g, wmu_ref[...], preferred_element_type=_f32) + bmu_ref[...]
    z = jnp.dot(g, wsg_ref[...], preferred_element_type=_f32) + bsg_ref[...]
    sg_ref[...] = jnp.maximum(z, 0.0) + jnp.log(1.0 + jnp.exp(-jnp.abs(z)))


def _sc_edge_body(h_hbm, as_hbm, ad_hbm, mb_hbm, src_hbm, dst_hbm,
                  osum_hbm, den_hbm,
                  as_v, ad_v, m_v, srcst, dstst, wbuf, den_v, rows, sidx, didx,
                  tmp_v, acc_v, osum_sh, dnm_sh, sem):
    core = lax.axis_index("c")
    sid = lax.axis_index("s")
    wid = sid * 2 + core
    base = sid * SEG

    pltpu.sync_copy(as_hbm, as_v)
    pltpu.sync_copy(ad_hbm, ad_v)
    pltpu.sync_copy(mb_hbm, m_v)
    pltpu.sync_copy(src_hbm.at[wid], srcst)
    pltpu.sync_copy(dst_hbm.at[wid], dstst)
    mv = m_v[pl.ds(0, 16)]

    zero16 = jnp.zeros((16,), _f32)

    def _zden(i, _):
        den_v[pl.ds(i * 16, 16)] = zero16
        return 0
    lax.fori_loop(0, NP // 16, _zden, 0)

    def _zrow(j, _):
        for q in range(4):
            rows[j, pl.ds(q * 16, 16)] = zero16
        return 0
    lax.fori_loop(0, CH, _zrow, 0)
    for t in range(SEG // CH):
        pltpu.sync_copy(rows, osum_sh.at[pl.ds(base + t * CH, CH)])

    def _wrow(r, _):
        def _wvec(v, _):
            s_vec = srcst[r, pl.ds(v * 16, 16)]
            d_vec = dstst[r, pl.ds(v * 16, 16)]
            a_sv = plsc.load_gather(as_v, [s_vec])
            a_dv = plsc.load_gather(ad_v, [d_vec])
            t1 = a_sv + a_dv
            t2 = mv + a_dv
            w = jnp.exp(jnp.maximum(t1, 0.2 * t1) - jnp.maximum(t2, 0.2 * t2))
            wbuf[r, pl.ds(v * 16, 16)] = w
            plsc.addupdate_scatter(den_v, [d_vec], w)
            return 0
        lax.fori_loop(0, CH // 16, _wvec, 0)
        return 0
    lax.fori_loop(0, NCHUNK, _wrow, 0)

    plsc.subcore_barrier()

    def _chunk(c, _):
        pltpu.sync_copy(src_hbm.at[wid, c], sidx)
        pltpu.async_copy(h_hbm.at[sidx], rows, sem).wait()

        def _scale(g, _):
            wv = wbuf[c, pl.ds(g * 16, 16)]
            for i in range(16):
                w = wv[i]
                j = g * 16 + i
                for q in range(4):
                    rows[j, pl.ds(q * 16, 16)] = rows[j, pl.ds(q * 16, 16)] * w
            return 0
        lax.fori_loop(0, CH // 16, _scale, 0)

        pltpu.sync_copy(dst_hbm.at[wid, c], didx)
        pltpu.sync_copy(rows, osum_sh.at[didx], add=True)
        return 0
    lax.fori_loop(0, NCHUNK, _chunk, 0)

    pltpu.sync_copy(den_v, dnm_sh.at[sid])
    plsc.subcore_barrier()

    def _zacc(i, _):
        acc_v[pl.ds(i * 16, 16)] = zero16
        return 0
    lax.fori_loop(0, SEG // 16, _zacc, 0)
    for k in range(16):
        pltpu.sync_copy(dnm_sh.at[k, pl.ds(base, SEG)], tmp_v)

        def _acc(i, _):
            acc_v[pl.ds(i * 16, 16)] = acc_v[pl.ds(i * 16, 16)] + tmp_v[pl.ds(i * 16, 16)]
            return 0
        lax.fori_loop(0, SEG // 16, _acc, 0)
    pltpu.sync_copy(acc_v, den_hbm.at[core, pl.ds(base, SEG)])

    for t in range(SEG // CH):
        pltpu.sync_copy(osum_sh.at[pl.ds(base + t * CH, CH)], rows)
        pltpu.sync_copy(rows, osum_hbm.at[core, pl.ds(base + t * CH, CH)])


_sc_edge = functools.partial(
    pl.kernel,
    out_type=[jax.ShapeDtypeStruct((2, NP, D_H), _f32),
              jax.ShapeDtypeStruct((2, NP), _f32)],
    mesh=plsc.VectorSubcoreMesh(core_axis_name="c", subcore_axis_name="s"),
    compiler_params=pltpu.CompilerParams(needs_layout_passes=False,
                                         use_tc_tiling_on_sc=False),
    scratch_types=[
        pltpu.VMEM((NP,), _f32),
        pltpu.VMEM((NP,), _f32),
        pltpu.VMEM((128,), _f32),
        pltpu.VMEM((NCHUNK, CH), jnp.int32),
        pltpu.VMEM((NCHUNK, CH), jnp.int32),
        pltpu.VMEM((NCHUNK, CH), _f32),
        pltpu.VMEM((NP,), _f32),
        pltpu.VMEM((CH, D_H), _f32),
        pltpu.VMEM((CH,), jnp.int32),
        pltpu.VMEM((CH,), jnp.int32),
        pltpu.VMEM((SEG,), _f32),
        pltpu.VMEM((SEG,), _f32),
        pltpu.VMEM_SHARED((NP, D_H), _f32),
        pltpu.VMEM_SHARED((16, NP), _f32),
        pltpu.SemaphoreType.DMA,
    ],
)(_sc_edge_body)


def kernel(x, edge_index, batch, W1, a1_src, a1_dst, b1, W2, a2_src, a2_dst,
           b2, W_mu, b_mu, W_sigma, b_sigma):
    xp = jnp.pad(x, ((0, NP - N), (0, 0)))
    batch_p = jnp.pad(batch, (0, NP - N), constant_values=NGRAPH)
    src3 = edge_index[0].reshape(NT, NCHUNK, CH)
    dst3 = edge_index[1].reshape(NT, NCHUNK, CH)

    h1, as1, ad1, mb1 = pl.pallas_call(
        _tc1_body,
        out_shape=[jax.ShapeDtypeStruct((NP, D_H), _f32),
                   jax.ShapeDtypeStruct((NP,), _f32),
                   jax.ShapeDtypeStruct((NP,), _f32),
                   jax.ShapeDtypeStruct((128,), _f32)],
    )(xp, W1, a1_src, a1_dst)

    osum1, den1 = _sc_edge(h1, as1, ad1, mb1, src3, dst3)

    h2, as2, ad2, mb2 = pl.pallas_call(
        _tc2_body,
        out_shape=[jax.ShapeDtypeStruct((NP, D_H), _f32),
                   jax.ShapeDtypeStruct((NP,), _f32),
                   jax.ShapeDtypeStruct((NP,), _f32),
                   jax.ShapeDtypeStruct((128,), _f32)],
    )(osum1, den1, h1, as1, ad1, b1, W2, a2_src, a2_dst)

    osum2, den2 = _sc_edge(h2, as2, ad2, mb2, src3, dst3)

    mu, sigma = pl.pallas_call(
        _tc3_body,
        out_shape=[jax.ShapeDtypeStruct((NGRAPH, SEQ_OUT * OUT_DIM), _f32),
                   jax.ShapeDtypeStruct((NGRAPH, SEQ_OUT * OUT_DIM), _f32)],
    )(osum2, den2, h2, as2, ad2, b2, batch_p, W_mu, b_mu, W_sigma, b_sigma)

    return (mu.reshape(NGRAPH, SEQ_OUT, OUT_DIM),
            sigma.reshape(NGRAPH, SEQ_OUT, OUT_DIM))


# speedup vs baseline: 44.5558x; 1.9952x over previous
import functools

import jax
import jax.numpy as jnp
from jax import lax
from jax.experimental import pallas as pl
from jax.experimental.pallas import tpu as pltpu
from jax.experimental.pallas import tpu_sc as plsc

N = 10000
NP = 10240
E = 320000
D_IN = 128
D_H = 64
OUT_DIM = 14
SEQ_OUT = 12
NGRAPH = 32

NT = 32
EPT = E // NT
NCHUNK = 125
CH = EPT // NCHUNK
SEG = NP // 16

_f32 = jnp.float32


def _lrelu(x):
    return jnp.maximum(x, 0.2 * x)


def _tc1_body(x_ref, w1_ref, a1s_ref, a1d_ref, h_ref, as_ref, ad_ref, mb_ref):
    h = jnp.dot(x_ref[...], w1_ref[...], preferred_element_type=_f32)
    h_ref[...] = h
    a_s = jnp.dot(h, a1s_ref[...], preferred_element_type=_f32)
    as_ref[...] = a_s
    ad_ref[...] = jnp.dot(h, a1d_ref[...], preferred_element_type=_f32)
    mb_ref[...] = jnp.full((128,), jnp.max(a_s), _f32)


def _agg(osum_ref, den_ref, h_ref, as_ref, ad_ref, b_ref):
    a_s = as_ref[...]
    a_d = ad_ref[...]
    m = jnp.max(a_s)
    ws = jnp.exp(_lrelu(a_s + a_d) - _lrelu(m + a_d))
    den = den_ref[0] + den_ref[1] + ws
    osum = osum_ref[0] + osum_ref[1] + ws[:, None] * h_ref[...]
    return jnp.maximum(osum / den[:, None] + b_ref[...], 0.0)


def _tc2_body(osum_ref, den_ref, h_ref, as_ref, ad_ref, b_ref, w2_ref,
              a2s_ref, a2d_ref, h2_ref, as2_ref, ad2_ref, mb_ref):
    hr = _agg(osum_ref, den_ref, h_ref, as_ref, ad_ref, b_ref)
    h2 = jnp.dot(hr, w2_ref[...], preferred_element_type=_f32)
    h2_ref[...] = h2
    a_s2 = jnp.dot(h2, a2s_ref[...], preferred_element_type=_f32)
    as2_ref[...] = a_s2
    ad2_ref[...] = jnp.dot(h2, a2d_ref[...], preferred_element_type=_f32)
    mb_ref[...] = jnp.full((128,), jnp.max(a_s2), _f32)


def _tc3_body(osum_ref, den_ref, h_ref, as_ref, ad_ref, b_ref, batch_ref,
              wmu_ref, bmu_ref, wsg_ref, bsg_ref, mu_ref, sg_ref):
    hr = _agg(osum_ref, den_ref, h_ref, as_ref, ad_ref, b_ref)
    gids = lax.broadcasted_iota(jnp.int32, (NP, NGRAPH), 1)
    oh = (batch_ref[...][:, None] == gids).astype(_f32)
    gsum = lax.dot_general(oh, hr, (((0,), (0,)), ((), ())),
                           preferred_element_type=_f32)
    cnt = jnp.sum(oh, axis=0)
    g = gsum / jnp.maximum(cnt, 1.0)[:, None]
    mu_ref[...] = jnp.dot(g, wmu_ref[...], preferred_element_type=_f32) + bmu_ref[...]
    z = jnp.dot(g, wsg_ref[...], preferred_element_type=_f32) + bsg_ref[...]
    sg_ref[...] = jnp.maximum(z, 0.0) + jnp.log(1.0 + jnp.exp(-jnp.abs(z)))


NBUF = 5


def _sc_edge_body(h_hbm, as_hbm, ad_hbm, mb_hbm, src_hbm, dst_hbm,
                  osum_hbm, den_hbm,
                  as_v, ad_v, m_v, srcst, dstst, wbuf,
                  rows0, rows1, rows2, rows3, rows4,
                  tmp_v, osum_sh, den_sh,
                  sg0, sg1, sg2, sg3, sg4, ss0, ss1, ss2, ss3, ss4):
    rows = [rows0, rows1, rows2, rows3, rows4]
    sg = [sg0, sg1, sg2, sg3, sg4]
    ss = [ss0, ss1, ss2, ss3, ss4]
    core = lax.axis_index("c")
    sid = lax.axis_index("s")
    wid = sid * 2 + core
    base = sid * SEG

    pltpu.sync_copy(as_hbm, as_v)
    pltpu.sync_copy(ad_hbm, ad_v)
    pltpu.sync_copy(mb_hbm, m_v)
    pltpu.sync_copy(src_hbm.at[wid], srcst)
    pltpu.sync_copy(dst_hbm.at[wid], dstst)
    mv = m_v[pl.ds(0, 16)]

    zero16 = jnp.zeros((16,), _f32)

    def _ztmp(i, _):
        tmp_v[pl.ds(i * 16, 16)] = zero16
        return 0
    lax.fori_loop(0, SEG // 16, _ztmp, 0)
    pltpu.sync_copy(tmp_v, den_sh.at[pl.ds(base, SEG)])

    def _zrow(j, _):
        for q in range(4):
            rows0[j, pl.ds(q * 16, 16)] = zero16
        return 0
    lax.fori_loop(0, CH, _zrow, 0)
    for t in range(SEG // CH):
        pltpu.sync_copy(rows0, osum_sh.at[pl.ds(base + t * CH, CH)])

    def _wrow(r, _):
        def _wvec(v, _):
            s_vec = srcst[r, pl.ds(v * 16, 16)]
            d_vec = dstst[r, pl.ds(v * 16, 16)]
            a_sv = plsc.load_gather(as_v, [s_vec])
            a_dv = plsc.load_gather(ad_v, [d_vec])
            t1 = a_sv + a_dv
            t2 = mv + a_dv
            w = jnp.exp(jnp.maximum(t1, 0.2 * t1) - jnp.maximum(t2, 0.2 * t2))
            wbuf[r, pl.ds(v * 16, 16)] = w
            return 0
        lax.fori_loop(0, CH // 16, _wvec, 0)
        return 0
    lax.fori_loop(0, NCHUNK, _wrow, 0)

    plsc.subcore_barrier()

    def _gather_start(c, b):
        pltpu.async_copy(h_hbm.at[srcst.at[c]], rows[b], sg[b])

    def _gather_wait(c, b):
        pltpu.make_async_copy(h_hbm.at[srcst.at[c]], rows[b], sg[b]).wait()

    def _scatter_start(c, b):
        pltpu.async_copy(rows[b], osum_sh.at[dstst.at[c]], ss[b], add=True)
        pltpu.async_copy(wbuf.at[c], den_sh.at[dstst.at[c]], ss[b], add=True)

    def _scatter_wait(c, b):
        pltpu.make_async_copy(rows[b], osum_sh.at[dstst.at[c]], ss[b]).wait()
        pltpu.make_async_copy(wbuf.at[c], den_sh.at[dstst.at[c]], ss[b]).wait()

    for b in range(3):
        _gather_start(b, b)

    n_outer = NCHUNK // NBUF

    def _outer(g, _):
        for b in range(NBUF):
            c = NBUF * g + b
            _gather_wait(c, b)

            def _scale(q, _):
                wv = wbuf[c, pl.ds(q * 16, 16)]
                for i in range(16):
                    w = wv[i]
                    j = q * 16 + i
                    for t in range(4):
                        rows[b][j, pl.ds(t * 16, 16)] = rows[b][j, pl.ds(t * 16, 16)] * w
                return 0
            lax.fori_loop(0, CH // 16, _scale, 0)

            _scatter_start(c, b)

            b3 = (b + 3) % NBUF
            if b < 2:
                @pl.when(g > 0)
                def _():
                    _scatter_wait(c - 2, b3)
                _gather_start(c + 3, b3)
            else:
                _scatter_wait(c - 2, b3)

                @pl.when(g < n_outer - 1)
                def _():
                    _gather_start(c + 3, b3)
        return 0
    lax.fori_loop(0, n_outer, _outer, 0)
    _scatter_wait(NCHUNK - 2, (NCHUNK - 2) % NBUF)
    _scatter_wait(NCHUNK - 1, (NCHUNK - 1) % NBUF)

    plsc.subcore_barrier()

    pltpu.sync_copy(den_sh.at[pl.ds(base, SEG)], tmp_v)
    pltpu.sync_copy(tmp_v, den_hbm.at[core, pl.ds(base, SEG)])

    for t in range(SEG // CH):
        pltpu.sync_copy(osum_sh.at[pl.ds(base + t * CH, CH)], rows0)
        pltpu.sync_copy(rows0, osum_hbm.at[core, pl.ds(base + t * CH, CH)])


_sc_edge = functools.partial(
    pl.kernel,
    out_type=[jax.ShapeDtypeStruct((2, NP, D_H), _f32),
              jax.ShapeDtypeStruct((2, NP), _f32)],
    mesh=plsc.VectorSubcoreMesh(core_axis_name="c", subcore_axis_name="s"),
    compiler_params=pltpu.CompilerParams(needs_layout_passes=False,
                                         use_tc_tiling_on_sc=False),
    scratch_types=[
        pltpu.VMEM((NP,), _f32),
        pltpu.VMEM((NP,), _f32),
        pltpu.VMEM((128,), _f32),
        pltpu.VMEM((NCHUNK, CH), jnp.int32),
        pltpu.VMEM((NCHUNK, CH), jnp.int32),
        pltpu.VMEM((NCHUNK, CH), _f32),
        pltpu.VMEM((CH, D_H), _f32),
        pltpu.VMEM((CH, D_H), _f32),
        pltpu.VMEM((CH, D_H), _f32),
        pltpu.VMEM((CH, D_H), _f32),
        pltpu.VMEM((CH, D_H), _f32),
        pltpu.VMEM((SEG,), _f32),
        pltpu.VMEM_SHARED((NP, D_H), _f32),
        pltpu.VMEM_SHARED((NP,), _f32),
    ] + [pltpu.SemaphoreType.DMA] * 10,
)(_sc_edge_body)


def kernel(x, edge_index, batch, W1, a1_src, a1_dst, b1, W2, a2_src, a2_dst,
           b2, W_mu, b_mu, W_sigma, b_sigma):
    xp = jnp.pad(x, ((0, NP - N), (0, 0)))
    batch_p = jnp.pad(batch, (0, NP - N), constant_values=NGRAPH)
    src3 = edge_index[0].reshape(NT, NCHUNK, CH)
    dst3 = edge_index[1].reshape(NT, NCHUNK, CH)

    h1, as1, ad1, mb1 = pl.pallas_call(
        _tc1_body,
        out_shape=[jax.ShapeDtypeStruct((NP, D_H), _f32),
                   jax.ShapeDtypeStruct((NP,), _f32),
                   jax.ShapeDtypeStruct((NP,), _f32),
                   jax.ShapeDtypeStruct((128,), _f32)],
    )(xp, W1, a1_src, a1_dst)

    osum1, den1 = _sc_edge(h1, as1, ad1, mb1, src3, dst3)

    h2, as2, ad2, mb2 = pl.pallas_call(
        _tc2_body,
        out_shape=[jax.ShapeDtypeStruct((NP, D_H), _f32),
                   jax.ShapeDtypeStruct((NP,), _f32),
                   jax.ShapeDtypeStruct((NP,), _f32),
                   jax.ShapeDtypeStruct((128,), _f32)],
    )(osum1, den1, h1, as1, ad1, b1, W2, a2_src, a2_dst)

    osum2, den2 = _sc_edge(h2, as2, ad2, mb2, src3, dst3)

    mu, sigma = pl.pallas_call(
        _tc3_body,
        out_shape=[jax.ShapeDtypeStruct((NGRAPH, SEQ_OUT * OUT_DIM), _f32),
                   jax.ShapeDtypeStruct((NGRAPH, SEQ_OUT * OUT_DIM), _f32)],
    )(osum2, den2, h2, as2, ad2, b2, batch_p, W_mu, b_mu, W_sigma, b_sigma)

    return (mu.reshape(NGRAPH, SEQ_OUT, OUT_DIM),
            sigma.reshape(NGRAPH, SEQ_OUT, OUT_DIM))


# speedup vs baseline: 44.5738x; 1.0004x over previous
import functools

import jax
import jax.numpy as jnp
from jax import lax
from jax.experimental import pallas as pl
from jax.experimental.pallas import tpu as pltpu
from jax.experimental.pallas import tpu_sc as plsc

N = 10000
NP = 10240
E = 320000
D_IN = 128
D_H = 64
OUT_DIM = 14
SEQ_OUT = 12
NGRAPH = 32

NT = 32
EPT = E // NT
NCHUNK = 125
CH = EPT // NCHUNK
SEG = NP // 16

_f32 = jnp.float32


def _lrelu(x):
    return jnp.maximum(x, 0.2 * x)


def _tc1_body(x_ref, w1_ref, a1s_ref, a1d_ref, h_ref, as_ref, ad_ref, mb_ref):
    h = jnp.dot(x_ref[...], w1_ref[...], preferred_element_type=_f32)
    h_ref[...] = h
    a_s = jnp.dot(h, a1s_ref[...], preferred_element_type=_f32)
    as_ref[...] = a_s
    ad_ref[...] = jnp.dot(h, a1d_ref[...], preferred_element_type=_f32)
    mb_ref[...] = jnp.full((128,), jnp.max(a_s), _f32)


def _agg(osum_ref, den_ref, h_ref, as_ref, ad_ref, b_ref):
    a_s = as_ref[...]
    a_d = ad_ref[...]
    m = jnp.max(a_s)
    ws = jnp.exp(_lrelu(a_s + a_d) - _lrelu(m + a_d))
    den = den_ref[0] + den_ref[1] + ws
    osum = osum_ref[0] + osum_ref[1] + ws[:, None] * h_ref[...]
    return jnp.maximum(osum / den[:, None] + b_ref[...], 0.0)


def _tc2_body(osum_ref, den_ref, h_ref, as_ref, ad_ref, b_ref, w2_ref,
              a2s_ref, a2d_ref, h2_ref, as2_ref, ad2_ref, mb_ref):
    hr = _agg(osum_ref, den_ref, h_ref, as_ref, ad_ref, b_ref)
    h2 = jnp.dot(hr, w2_ref[...], preferred_element_type=_f32)
    h2_ref[...] = h2
    a_s2 = jnp.dot(h2, a2s_ref[...], preferred_element_type=_f32)
    as2_ref[...] = a_s2
    ad2_ref[...] = jnp.dot(h2, a2d_ref[...], preferred_element_type=_f32)
    mb_ref[...] = jnp.full((128,), jnp.max(a_s2), _f32)


def _tc3_body(osum_ref, den_ref, h_ref, as_ref, ad_ref, b_ref, batch_ref,
              wmu_ref, bmu_ref, wsg_ref, bsg_ref, mu_ref, sg_ref):
    hr = _agg(osum_ref, den_ref, h_ref, as_ref, ad_ref, b_ref)
    gids = lax.broadcasted_iota(jnp.int32, (NP, NGRAPH), 1)
    oh = (batch_ref[...][:, None] == gids).astype(_f32)
    gsum = lax.dot_general(oh, hr, (((0,), (0,)), ((), ())),
                           preferred_element_type=_f32)
    cnt = jnp.sum(oh, axis=0)
    g = gsum / jnp.maximum(cnt, 1.0)[:, None]
    mu_ref[...] = jnp.dot(g, wmu_ref[...], preferred_element_type=_f32) + bmu_ref[...]
    z = jnp.dot(g, wsg_ref[...], preferred_element_type=_f32) + bsg_ref[...]
    sg_ref[...] = jnp.maximum(z, 0.0) + jnp.log(1.0 + jnp.exp(-jnp.abs(z)))


NBUF = 5


def _sc_edge_body(h_hbm, as_hbm, ad_hbm, mb_hbm, src_hbm, dst_hbm,
                  osum_hbm, den_hbm,
                  as_v, ad_v, m_v, srcst, dstst, wbuf,
                  rows0, rows1, rows2, rows3, rows4,
                  tmp_v, osum_sh, den_sh,
                  sg0, sg1, sg2, sg3, sg4, ss0, ss1, ss2, ss3, ss4):
    rows = [rows0, rows1, rows2, rows3, rows4]
    sg = [sg0, sg1, sg2, sg3, sg4]
    ss = [ss0, ss1, ss2, ss3, ss4]
    core = lax.axis_index("c")
    sid = lax.axis_index("s")
    wid = sid * 2 + core
    base = sid * SEG

    pltpu.sync_copy(as_hbm, as_v)
    pltpu.sync_copy(ad_hbm, ad_v)
    pltpu.sync_copy(mb_hbm, m_v)
    pltpu.sync_copy(src_hbm.at[wid], srcst)
    pltpu.sync_copy(dst_hbm.at[wid], dstst)
    mv = m_v[pl.ds(0, 16)]

    zero16 = jnp.zeros((16,), _f32)

    def _ztmp(i, _):
        tmp_v[pl.ds(i * 16, 16)] = zero16
        return 0
    lax.fori_loop(0, SEG // 16, _ztmp, 0)
    pltpu.sync_copy(tmp_v, den_sh.at[pl.ds(base, SEG)])

    def _zrow(j, _):
        for q in range(4):
            rows0[j, pl.ds(q * 16, 16)] = zero16
        return 0
    lax.fori_loop(0, CH, _zrow, 0)
    for t in range(SEG // CH):
        pltpu.sync_copy(rows0, osum_sh.at[pl.ds(base + t * CH, CH)])

    _ws = jax.named_scope("wpass"); _ws.__enter__()
    def _wrow(r, _):
        def _wvec(v, _):
            s_vec = srcst[r, pl.ds(v * 16, 16)]
            d_vec = dstst[r, pl.ds(v * 16, 16)]
            a_sv = plsc.load_gather(as_v, [s_vec])
            a_dv = plsc.load_gather(ad_v, [d_vec])
            t1 = a_sv + a_dv
            t2 = mv + a_dv
            w = jnp.exp(jnp.maximum(t1, 0.2 * t1) - jnp.maximum(t2, 0.2 * t2))
            wbuf[r, pl.ds(v * 16, 16)] = w
            return 0
        lax.fori_loop(0, CH // 16, _wvec, 0)
        return 0
    lax.fori_loop(0, NCHUNK, _wrow, 0)
    _ws.__exit__(None, None, None)

    plsc.subcore_barrier()
    _rp = jax.named_scope("rowphase"); _rp.__enter__()

    def _gather_start(c, b):
        pltpu.async_copy(h_hbm.at[srcst.at[c]], rows[b], sg[b])

    def _gather_wait(c, b):
        pltpu.make_async_copy(h_hbm.at[srcst.at[c]], rows[b], sg[b]).wait()

    def _scatter_start(c, b):
        pltpu.async_copy(rows[b], osum_sh.at[dstst.at[c]], ss[b], add=True)
        pltpu.async_copy(wbuf.at[c], den_sh.at[dstst.at[c]], ss[b], add=True)

    def _scatter_wait(c, b):
        pltpu.make_async_copy(rows[b], osum_sh.at[dstst.at[c]], ss[b]).wait()
        pltpu.make_async_copy(wbuf.at[c], den_sh.at[dstst.at[c]], ss[b]).wait()

    for b in range(3):
        _gather_start(b, b)

    n_outer = NCHUNK // NBUF

    def _outer(g, _):
        for b in range(NBUF):
            c = NBUF * g + b
            _gather_wait(c, b)

            def _scale(q, _):
                wv = wbuf[c, pl.ds(q * 16, 16)]
                for i in range(16):
                    w = wv[i]
                    j = q * 16 + i
                    for t in range(4):
                        rows[b][j, pl.ds(t * 16, 16)] = rows[b][j, pl.ds(t * 16, 16)] * w
                return 0
            lax.fori_loop(0, CH // 16, _scale, 0)

            _scatter_start(c, b)

            b3 = (b + 3) % NBUF
            if b < 2:
                @pl.when(g > 0)
                def _():
                    _scatter_wait(c - 2, b3)
                _gather_start(c + 3, b3)
            else:
                _scatter_wait(c - 2, b3)

                @pl.when(g < n_outer - 1)
                def _():
                    _gather_start(c + 3, b3)
        return 0
    lax.fori_loop(0, n_outer, _outer, 0)
    _scatter_wait(NCHUNK - 2, (NCHUNK - 2) % NBUF)
    _scatter_wait(NCHUNK - 1, (NCHUNK - 1) % NBUF)
    _rp.__exit__(None, None, None)

    plsc.subcore_barrier()
    _wb = jax.named_scope("writeback"); _wb.__enter__()

    pltpu.sync_copy(den_sh.at[pl.ds(base, SEG)], tmp_v)
    pltpu.sync_copy(tmp_v, den_hbm.at[core, pl.ds(base, SEG)])

    for t in range(SEG // CH):
        pltpu.sync_copy(osum_sh.at[pl.ds(base + t * CH, CH)], rows0)
        pltpu.sync_copy(rows0, osum_hbm.at[core, pl.ds(base + t * CH, CH)])
    _wb.__exit__(None, None, None)


_sc_edge = functools.partial(
    pl.kernel,
    out_type=[jax.ShapeDtypeStruct((2, NP, D_H), _f32),
              jax.ShapeDtypeStruct((2, NP), _f32)],
    mesh=plsc.VectorSubcoreMesh(core_axis_name="c", subcore_axis_name="s"),
    compiler_params=pltpu.CompilerParams(needs_layout_passes=False,
                                         use_tc_tiling_on_sc=False),
    scratch_types=[
        pltpu.VMEM((NP,), _f32),
        pltpu.VMEM((NP,), _f32),
        pltpu.VMEM((128,), _f32),
        pltpu.VMEM((NCHUNK, CH), jnp.int32),
        pltpu.VMEM((NCHUNK, CH), jnp.int32),
        pltpu.VMEM((NCHUNK, CH), _f32),
        pltpu.VMEM((CH, D_H), _f32),
        pltpu.VMEM((CH, D_H), _f32),
        pltpu.VMEM((CH, D_H), _f32),
        pltpu.VMEM((CH, D_H), _f32),
        pltpu.VMEM((CH, D_H), _f32),
        pltpu.VMEM((SEG,), _f32),
        pltpu.VMEM_SHARED((NP, D_H), _f32),
        pltpu.VMEM_SHARED((NP,), _f32),
    ] + [pltpu.SemaphoreType.DMA] * 10,
)(_sc_edge_body)


def kernel(x, edge_index, batch, W1, a1_src, a1_dst, b1, W2, a2_src, a2_dst,
           b2, W_mu, b_mu, W_sigma, b_sigma):
    xp = jnp.pad(x, ((0, NP - N), (0, 0)))
    batch_p = jnp.pad(batch, (0, NP - N), constant_values=NGRAPH)
    src3 = edge_index[0].reshape(NT, NCHUNK, CH)
    dst3 = edge_index[1].reshape(NT, NCHUNK, CH)

    h1, as1, ad1, mb1 = pl.pallas_call(
        _tc1_body,
        out_shape=[jax.ShapeDtypeStruct((NP, D_H), _f32),
                   jax.ShapeDtypeStruct((NP,), _f32),
                   jax.ShapeDtypeStruct((NP,), _f32),
                   jax.ShapeDtypeStruct((128,), _f32)],
    )(xp, W1, a1_src, a1_dst)

    osum1, den1 = _sc_edge(h1, as1, ad1, mb1, src3, dst3)

    h2, as2, ad2, mb2 = pl.pallas_call(
        _tc2_body,
        out_shape=[jax.ShapeDtypeStruct((NP, D_H), _f32),
                   jax.ShapeDtypeStruct((NP,), _f32),
                   jax.ShapeDtypeStruct((NP,), _f32),
                   jax.ShapeDtypeStruct((128,), _f32)],
    )(osum1, den1, h1, as1, ad1, b1, W2, a2_src, a2_dst)

    osum2, den2 = _sc_edge(h2, as2, ad2, mb2, src3, dst3)

    mu, sigma = pl.pallas_call(
        _tc3_body,
        out_shape=[jax.ShapeDtypeStruct((NGRAPH, SEQ_OUT * OUT_DIM), _f32),
                   jax.ShapeDtypeStruct((NGRAPH, SEQ_OUT * OUT_DIM), _f32)],
    )(osum2, den2, h2, as2, ad2, b2, batch_p, W_mu, b_mu, W_sigma, b_sigma)

    return (mu.reshape(NGRAPH, SEQ_OUT, OUT_DIM),
            sigma.reshape(NGRAPH, SEQ_OUT, OUT_DIM))
